# baseline (speedup 1.0000x reference)
import jax
import jax.numpy as jnp
from jax import lax
from jax.experimental import pallas as pl
from jax.experimental.pallas import tpu as pltpu
from jax.experimental.pallas import tpu_sc as plsc

_BATCH = 16384
_HIST = 20
_DIM = 64
_NC = 2
_NS = 16
_NW = _NC * _NS
_ROWS_W = _BATCH // _NW
_S = 16
_NBLK = _ROWS_W // _S
_NBUF = 4
_LAG = 2


def _gather_body(idx_hbm, table_hbm, out_hbm, idx_v, stage, gsem, ssem):
    wid = lax.axis_index("s") * _NC + lax.axis_index("c")
    row0 = wid * _ROWS_W
    pltpu.sync_copy(idx_hbm.at[pl.ds(row0, _ROWS_W)], idx_v)

    def fire_gathers(k, b):
        for i in range(_S):
            pltpu.async_copy(
                table_hbm.at[idx_v.at[k * _S + i]], stage.at[b, i], gsem.at[b])

    def drain_gathers(k, b):
        for i in range(_S):
            pltpu.make_async_copy(
                table_hbm.at[idx_v.at[k * _S + i]], stage.at[b, i],
                gsem.at[b]).wait()

    def issue_scatter(k, b):
        pltpu.async_copy(
            stage.at[b], out_hbm.at[pl.ds(row0 + k * _S, _S)], ssem.at[b])

    def wait_scatter(k, b):
        pltpu.make_async_copy(
            stage.at[b], out_hbm.at[pl.ds(row0 + k * _S, _S)],
            ssem.at[b]).wait()

    for b in range(_LAG):
        fire_gathers(b, b)

    def step(k0, carry):
        for bi in range(_NBUF):
            k = k0 + bi
            b = bi
            drain_gathers(k, b)
            issue_scatter(k, b)
            bn = (bi + _LAG) % _NBUF

            @pl.when(k + _LAG < _NBLK)
            def _():
                @pl.when(k >= _LAG)
                def _():
                    wait_scatter(k - _LAG, bn)
                fire_gathers(k + _LAG, bn)

        return carry

    lax.fori_loop(0, _NBLK // _NBUF, lambda i, c: step(i * _NBUF, c), 0)

    for m in range(_NBLK - 2 * _LAG, _NBLK):
        wait_scatter(m, m % _NBUF)


@jax.jit
def kernel(indices, table):
    idx = indices.astype(jnp.int32)
    mesh = plsc.VectorSubcoreMesh(core_axis_name="c", subcore_axis_name="s")
    out = pl.kernel(
        _gather_body,
        out_type=jax.ShapeDtypeStruct((_BATCH, _HIST, _DIM), jnp.float32),
        mesh=mesh,
        scratch_types=[
            pltpu.VMEM((_ROWS_W, _HIST), jnp.int32),
            pltpu.VMEM((_NBUF, _S, _HIST, _DIM), jnp.float32),
            pltpu.SemaphoreType.DMA((_NBUF,)),
            pltpu.SemaphoreType.DMA((_NBUF,)),
        ],
        compiler_params=pltpu.CompilerParams(use_tc_tiling_on_sc=False),
    )(idx, table)
    return out
